# baseline (speedup 1.0000x reference)
import functools

import jax
import jax.numpy as jnp
from jax import lax
from jax.experimental import pallas as pl
from jax.experimental.pallas import tpu as pltpu
from jax.experimental.pallas import tpu_sc as plsc

K = 1024
D = 256
N = 18432
TILE = 1024
GRID = N // TILE
COMMITMENT_COST = 0.25


def _vq_tc_body(x_ref, w_ref, idx_ref, scal_ref, counts_acc, dsum_acc):
    step = pl.program_id(0)

    @pl.when(step == 0)
    def _init():
        counts_acc[...] = jnp.zeros_like(counts_acc)
        dsum_acc[0] = 0.0

    x = x_ref[...]
    w = w_ref[...]
    xsq = jnp.sum(x * x, axis=1, keepdims=True)
    wsq = jnp.sum(w * w, axis=1)
    s = lax.dot_general(x, w, (((1,), (1,)), ((), ())),
                        preferred_element_type=jnp.float32)
    d = xsq - 2.0 * s + wsq[None, :]
    dmin = jnp.min(d, axis=1, keepdims=True)
    iota = lax.broadcasted_iota(jnp.int32, (TILE, K), 1)
    idx = jnp.min(jnp.where(d == dmin, iota, jnp.int32(K)), axis=1)
    idx_ref[...] = idx

    onehot = (iota == idx[:, None]).astype(jnp.float32)
    counts_acc[...] += jnp.sum(onehot, axis=0, keepdims=True)
    dsum_acc[0] += jnp.sum(dmin)

    @pl.when(step == GRID - 1)
    def _epilogue():
        counts = counts_acc[...]
        p = counts / jnp.float32(N)
        ent = jnp.sum(p * jnp.log(p + 1e-10))
        perplexity = jnp.exp(-ent)
        usage = jnp.sum((counts > 0.0).astype(jnp.float32)) / jnp.float32(K)
        mse = dsum_acc[0] / jnp.float32(N * D)
        scal_ref[0] = mse + COMMITMENT_COST * mse
        scal_ref[1] = mse
        scal_ref[2] = mse
        scal_ref[3] = perplexity
        scal_ref[4] = usage


_vq_tc = pl.pallas_call(
    _vq_tc_body,
    grid=(GRID,),
    in_specs=[
        pl.BlockSpec((TILE, D), lambda i: (i, 0)),
        pl.BlockSpec((K, D), lambda i: (0, 0)),
    ],
    out_specs=[
        pl.BlockSpec((TILE,), lambda i: (i,)),
        pl.BlockSpec(memory_space=pltpu.SMEM),
    ],
    out_shape=[
        jax.ShapeDtypeStruct((N,), jnp.int32),
        jax.ShapeDtypeStruct((8,), jnp.float32),
    ],
    scratch_shapes=[
        pltpu.VMEM((1, K), jnp.float32),
        pltpu.SMEM((8,), jnp.float32),
    ],
)


_NC, _NS = 2, 16
_NW = _NC * _NS
_BPW = N // _NW
_CHUNK = 96
_NCHUNK = _BPW // _CHUNK


@functools.cache
def _make_gather():
    @functools.partial(
        pl.kernel,
        mesh=plsc.VectorSubcoreMesh(core_axis_name="c", subcore_axis_name="s",
                                    num_cores=_NC, num_subcores=_NS),
        out_type=jax.ShapeDtypeStruct((N, D), jnp.float32),
        scratch_types=[
            pltpu.VMEM((_CHUNK,), jnp.int32),
            pltpu.VMEM((_CHUNK, D), jnp.float32),
            pltpu.SemaphoreType.DMA,
        ],
    )
    def _gather_rows(table_hbm, idx_hbm, out_hbm, idx_v, rows_v, sem):
        wid = lax.axis_index("s") * _NC + lax.axis_index("c")
        base = wid * _BPW
        for c in range(_NCHUNK):
            off = base + c * _CHUNK
            pltpu.sync_copy(idx_hbm.at[pl.ds(off, _CHUNK)], idx_v)
            pltpu.async_copy(table_hbm.at[idx_v], rows_v, sem).wait()
            pltpu.sync_copy(rows_v, out_hbm.at[pl.ds(off, _CHUNK)])

    return _gather_rows


def kernel(inputs, weight):
    idx, scal = _vq_tc(inputs, weight)
    quantized = _make_gather()(weight, idx)
    return (quantized, idx, scal[0], scal[1], scal[2], scal[3], scal[4])


# speedup vs baseline: 1.1122x; 1.1122x over previous
import functools

import jax
import jax.numpy as jnp
from jax import lax
from jax.experimental import pallas as pl
from jax.experimental.pallas import tpu as pltpu
from jax.experimental.pallas import tpu_sc as plsc

K = 1024
D = 256
N = 18432
TILE = 1024
GRID = N // TILE
COMMITMENT_COST = 0.25


def _vq_tc_body(x_ref, w_ref, idx_ref, scal_ref, counts_acc, dsum_acc, wsq_acc):
    step = pl.program_id(0)

    @pl.when(step == 0)
    def _init():
        counts_acc[...] = jnp.zeros_like(counts_acc)
        dsum_acc[0] = 0.0
        w0 = w_ref[...]
        wsq_acc[...] = jnp.sum(w0 * w0, axis=1)[None, :]

    x = x_ref[...]
    w = w_ref[...]
    xsq = jnp.sum(x * x, axis=1, keepdims=True)
    wsq = wsq_acc[...]
    s = lax.dot_general(x, w, (((1,), (1,)), ((), ())),
                        preferred_element_type=jnp.float32)
    d = xsq - 2.0 * s + wsq
    dmin = jnp.min(d, axis=1, keepdims=True)
    iota_f = lax.broadcasted_iota(jnp.int32, (TILE, K), 1).astype(jnp.float32)
    idxf = jnp.min(jnp.where(d == dmin, iota_f, jnp.float32(K)), axis=1,
                   keepdims=True)
    idx_ref[...] = idxf[:, 0].astype(jnp.int32)

    onehot = jnp.where(iota_f == idxf, 1.0, 0.0)
    counts_acc[...] += jnp.sum(onehot, axis=0, keepdims=True)
    dsum_acc[0] += jnp.sum(dmin)

    @pl.when(step == GRID - 1)
    def _epilogue():
        counts = counts_acc[...]
        p = counts / jnp.float32(N)
        ent = jnp.sum(p * jnp.log(p + 1e-10))
        perplexity = jnp.exp(-ent)
        usage = jnp.sum((counts > 0.0).astype(jnp.float32)) / jnp.float32(K)
        mse = dsum_acc[0] / jnp.float32(N * D)
        scal_ref[0] = mse + COMMITMENT_COST * mse
        scal_ref[1] = mse
        scal_ref[2] = mse
        scal_ref[3] = perplexity
        scal_ref[4] = usage


_vq_tc = pl.pallas_call(
    _vq_tc_body,
    grid=(GRID,),
    in_specs=[
        pl.BlockSpec((TILE, D), lambda i: (i, 0)),
        pl.BlockSpec((K, D), lambda i: (0, 0)),
    ],
    out_specs=[
        pl.BlockSpec((TILE,), lambda i: (i,)),
        pl.BlockSpec(memory_space=pltpu.SMEM),
    ],
    out_shape=[
        jax.ShapeDtypeStruct((N,), jnp.int32),
        jax.ShapeDtypeStruct((8,), jnp.float32),
    ],
    scratch_shapes=[
        pltpu.VMEM((1, K), jnp.float32),
        pltpu.SMEM((8,), jnp.float32),
        pltpu.VMEM((1, K), jnp.float32),
    ],
)


_NC, _NS = 2, 16
_NW = _NC * _NS
_BPW = N // _NW
_CHUNK = 96
_NCHUNK = _BPW // _CHUNK


@functools.cache
def _make_gather():
    @functools.partial(
        pl.kernel,
        mesh=plsc.VectorSubcoreMesh(core_axis_name="c", subcore_axis_name="s",
                                    num_cores=_NC, num_subcores=_NS),
        out_type=jax.ShapeDtypeStruct((N, D), jnp.float32),
        scratch_types=[
            pltpu.VMEM((_CHUNK,), jnp.int32),
            pltpu.VMEM((_CHUNK, D), jnp.float32),
            pltpu.SemaphoreType.DMA,
        ],
    )
    def _gather_rows(table_hbm, idx_hbm, out_hbm, idx_v, rows_v, sem):
        wid = lax.axis_index("s") * _NC + lax.axis_index("c")
        base = wid * _BPW
        for c in range(_NCHUNK):
            off = base + c * _CHUNK
            pltpu.sync_copy(idx_hbm.at[pl.ds(off, _CHUNK)], idx_v)
            pltpu.async_copy(table_hbm.at[idx_v], rows_v, sem).wait()
            pltpu.sync_copy(rows_v, out_hbm.at[pl.ds(off, _CHUNK)])

    return _gather_rows


def kernel(inputs, weight):
    idx, scal = _vq_tc(inputs, weight)
    quantized = _make_gather()(weight, idx)
    return (quantized, idx, scal[0], scal[1], scal[2], scal[3], scal[4])


# speedup vs baseline: 1.1227x; 1.0095x over previous
import functools

import jax
import jax.numpy as jnp
from jax import lax
from jax.experimental import pallas as pl
from jax.experimental.pallas import tpu as pltpu
from jax.experimental.pallas import tpu_sc as plsc

K = 1024
D = 256
N = 18432
TILE = 1024
GRID = N // TILE
COMMITMENT_COST = 0.25


def _argmin_tc_body(x_ref, w_ref, idx_ref, dsum_ref, dsum_acc, wsq_acc):
    step = pl.program_id(0)

    @pl.when(step == 0)
    def _init():
        dsum_acc[0] = 0.0
        w0 = w_ref[...]
        wsq_acc[...] = jnp.sum(w0 * w0, axis=1)[None, :]

    x = x_ref[...]
    w = w_ref[...]
    xsq = jnp.sum(x * x, axis=1, keepdims=True)
    wsq = wsq_acc[...]
    s = lax.dot_general(x, w, (((1,), (1,)), ((), ())),
                        preferred_element_type=jnp.float32)
    d = xsq - 2.0 * s + wsq
    dmin = jnp.min(d, axis=1, keepdims=True)
    iota_f = lax.broadcasted_iota(jnp.int32, (TILE, K), 1).astype(jnp.float32)
    idxf = jnp.min(jnp.where(d == dmin, iota_f, jnp.float32(K)), axis=1,
                   keepdims=True)
    idx_ref[...] = idxf[:, 0].astype(jnp.int32)
    dsum_acc[0] += jnp.sum(dmin)

    @pl.when(step == GRID - 1)
    def _fin():
        dsum_ref[0] = dsum_acc[0]


_argmin_tc = pl.pallas_call(
    _argmin_tc_body,
    grid=(GRID,),
    in_specs=[
        pl.BlockSpec((TILE, D), lambda i: (i, 0)),
        pl.BlockSpec((K, D), lambda i: (0, 0)),
    ],
    out_specs=[
        pl.BlockSpec((TILE,), lambda i: (i,)),
        pl.BlockSpec(memory_space=pltpu.SMEM),
    ],
    out_shape=[
        jax.ShapeDtypeStruct((N,), jnp.int32),
        jax.ShapeDtypeStruct((1,), jnp.float32),
    ],
    scratch_shapes=[
        pltpu.SMEM((1,), jnp.float32),
        pltpu.VMEM((1, K), jnp.float32),
    ],
)


def _scalars_tc_body(idx_ref, dsum_ref, scal_ref, counts_acc):
    step = pl.program_id(0)

    @pl.when(step == 0)
    def _init():
        counts_acc[...] = jnp.zeros_like(counts_acc)

    idx = idx_ref[...]
    iota = lax.broadcasted_iota(jnp.int32, (TILE, K), 1)
    onehot = jnp.where(iota == idx[:, None], 1.0, 0.0)
    counts_acc[...] += jnp.sum(onehot, axis=0, keepdims=True)

    @pl.when(step == GRID - 1)
    def _epilogue():
        counts = counts_acc[...]
        p = counts / jnp.float32(N)
        ent = jnp.sum(p * jnp.log(p + 1e-10))
        perplexity = jnp.exp(-ent)
        usage = jnp.sum((counts > 0.0).astype(jnp.float32)) / jnp.float32(K)
        mse = dsum_ref[0] / jnp.float32(N * D)
        scal_ref[0] = mse + COMMITMENT_COST * mse
        scal_ref[1] = mse
        scal_ref[2] = mse
        scal_ref[3] = perplexity
        scal_ref[4] = usage


_scalars_tc = pl.pallas_call(
    _scalars_tc_body,
    grid=(GRID,),
    in_specs=[
        pl.BlockSpec((TILE,), lambda i: (i,)),
        pl.BlockSpec(memory_space=pltpu.SMEM),
    ],
    out_specs=pl.BlockSpec(memory_space=pltpu.SMEM),
    out_shape=jax.ShapeDtypeStruct((8,), jnp.float32),
    scratch_shapes=[
        pltpu.VMEM((1, K), jnp.float32),
    ],
)


_NC, _NS = 2, 16
_NW = _NC * _NS
_BPW = N // _NW
_CHUNK = 96
_NCHUNK = _BPW // _CHUNK


@functools.cache
def _make_gather():
    @functools.partial(
        pl.kernel,
        mesh=plsc.VectorSubcoreMesh(core_axis_name="c", subcore_axis_name="s",
                                    num_cores=_NC, num_subcores=_NS),
        out_type=jax.ShapeDtypeStruct((N, D), jnp.float32),
        scratch_types=[
            pltpu.VMEM((_CHUNK,), jnp.int32),
            pltpu.VMEM((_CHUNK, D), jnp.float32),
            pltpu.SemaphoreType.DMA,
        ],
    )
    def _gather_rows(table_hbm, idx_hbm, out_hbm, idx_v, rows_v, sem):
        wid = lax.axis_index("s") * _NC + lax.axis_index("c")
        base = wid * _BPW
        for c in range(_NCHUNK):
            off = base + c * _CHUNK
            pltpu.sync_copy(idx_hbm.at[pl.ds(off, _CHUNK)], idx_v)
            pltpu.async_copy(table_hbm.at[idx_v], rows_v, sem).wait()
            pltpu.sync_copy(rows_v, out_hbm.at[pl.ds(off, _CHUNK)])

    return _gather_rows


def kernel(inputs, weight):
    idx, dsum = _argmin_tc(inputs, weight)
    quantized = _make_gather()(weight, idx)
    scal = _scalars_tc(idx, dsum)
    return (quantized, idx, scal[0], scal[1], scal[2], scal[3], scal[4])


# speedup vs baseline: 1.1475x; 1.0220x over previous
import functools

import jax
import jax.numpy as jnp
from jax import lax
from jax.experimental import pallas as pl
from jax.experimental.pallas import tpu as pltpu
from jax.experimental.pallas import tpu_sc as plsc

K = 1024
D = 256
N = 18432
TILE = 6144
GRID = N // TILE
COMMITMENT_COST = 0.25


def _argmin_tc_body(x_ref, w_ref, idx_ref, dsum_ref, dsum_acc, wsq_acc):
    step = pl.program_id(0)

    @pl.when(step == 0)
    def _init():
        dsum_acc[0] = 0.0
        w0 = w_ref[...]
        wsq_acc[...] = jnp.sum(w0 * w0, axis=1)[None, :]

    x = x_ref[...]
    w = w_ref[...]
    xsq = jnp.sum(x * x, axis=1, keepdims=True)
    wsq = wsq_acc[...]
    s2 = lax.dot_general(x + x, w, (((1,), (1,)), ((), ())),
                         preferred_element_type=jnp.float32)
    d = (xsq - s2) + wsq
    dmin = jnp.min(d, axis=1, keepdims=True)
    iota_f = lax.broadcasted_iota(jnp.int32, (TILE, K), 1).astype(jnp.float32)
    idxf = jnp.min(jnp.where(d == dmin, iota_f, jnp.float32(K)), axis=1,
                   keepdims=True)
    idx_ref[...] = idxf[:, 0].astype(jnp.int32)
    dsum_acc[0] += jnp.sum(dmin)

    @pl.when(step == GRID - 1)
    def _fin():
        dsum_ref[0] = dsum_acc[0]


_argmin_tc = pl.pallas_call(
    _argmin_tc_body,
    grid=(GRID,),
    in_specs=[
        pl.BlockSpec((TILE, D), lambda i: (i, 0)),
        pl.BlockSpec((K, D), lambda i: (0, 0)),
    ],
    out_specs=[
        pl.BlockSpec((TILE,), lambda i: (i,)),
        pl.BlockSpec(memory_space=pltpu.SMEM),
    ],
    out_shape=[
        jax.ShapeDtypeStruct((N,), jnp.int32),
        jax.ShapeDtypeStruct((1,), jnp.float32),
    ],
    scratch_shapes=[
        pltpu.SMEM((1,), jnp.float32),
        pltpu.VMEM((1, K), jnp.float32),
    ],
)


def _scalars_tc_body(idx_ref, dsum_ref, scal_ref, counts_acc):
    step = pl.program_id(0)

    @pl.when(step == 0)
    def _init():
        counts_acc[...] = jnp.zeros_like(counts_acc)

    idx = idx_ref[...]
    iota = lax.broadcasted_iota(jnp.int32, (TILE, K), 1)
    onehot = jnp.where(iota == idx[:, None], 1.0, 0.0)
    counts_acc[...] += jnp.sum(onehot, axis=0, keepdims=True)

    @pl.when(step == GRID - 1)
    def _epilogue():
        counts = counts_acc[...]
        p = counts / jnp.float32(N)
        ent = jnp.sum(p * jnp.log(p + 1e-10))
        perplexity = jnp.exp(-ent)
        usage = jnp.sum((counts > 0.0).astype(jnp.float32)) / jnp.float32(K)
        mse = dsum_ref[0] / jnp.float32(N * D)
        scal_ref[0] = mse + COMMITMENT_COST * mse
        scal_ref[1] = mse
        scal_ref[2] = mse
        scal_ref[3] = perplexity
        scal_ref[4] = usage


_scalars_tc = pl.pallas_call(
    _scalars_tc_body,
    grid=(GRID,),
    in_specs=[
        pl.BlockSpec((TILE,), lambda i: (i,)),
        pl.BlockSpec(memory_space=pltpu.SMEM),
    ],
    out_specs=pl.BlockSpec(memory_space=pltpu.SMEM),
    out_shape=jax.ShapeDtypeStruct((8,), jnp.float32),
    scratch_shapes=[
        pltpu.VMEM((1, K), jnp.float32),
    ],
)


_NC, _NS = 2, 16
_NW = _NC * _NS
_BPW = N // _NW
_CHUNK = 96
_NCHUNK = _BPW // _CHUNK


@functools.cache
def _make_gather():
    @functools.partial(
        pl.kernel,
        mesh=plsc.VectorSubcoreMesh(core_axis_name="c", subcore_axis_name="s",
                                    num_cores=_NC, num_subcores=_NS),
        out_type=jax.ShapeDtypeStruct((N, D), jnp.float32),
        scratch_types=[
            pltpu.VMEM((_CHUNK,), jnp.int32),
            pltpu.VMEM((_CHUNK, D), jnp.float32),
            pltpu.SemaphoreType.DMA,
        ],
    )
    def _gather_rows(table_hbm, idx_hbm, out_hbm, idx_v, rows_v, sem):
        wid = lax.axis_index("s") * _NC + lax.axis_index("c")
        base = wid * _BPW
        for c in range(_NCHUNK):
            off = base + c * _CHUNK
            pltpu.sync_copy(idx_hbm.at[pl.ds(off, _CHUNK)], idx_v)
            pltpu.async_copy(table_hbm.at[idx_v], rows_v, sem).wait()
            pltpu.sync_copy(rows_v, out_hbm.at[pl.ds(off, _CHUNK)])

    return _gather_rows


def kernel(inputs, weight):
    idx, dsum = _argmin_tc(inputs, weight)
    quantized = _make_gather()(weight, idx)
    scal = _scalars_tc(idx, dsum)
    return (quantized, idx, scal[0], scal[1], scal[2], scal[3], scal[4])


# speedup vs baseline: 1.2146x; 1.0585x over previous
import functools

import jax
import jax.numpy as jnp
from jax import lax
from jax.experimental import pallas as pl
from jax.experimental.pallas import tpu as pltpu
from jax.experimental.pallas import tpu_sc as plsc

K = 1024
D = 256
N = 18432
TILE = 6144
GRID = N // TILE
COMMITMENT_COST = 0.25


def _argmin_tc_body(x_ref, w_ref, idx_ref, dsum_ref, dsum_acc, wsq_acc):
    step = pl.program_id(0)

    @pl.when(step == 0)
    def _init():
        dsum_acc[0] = 0.0
        w0 = w_ref[...]
        wsq_acc[...] = jnp.sum(w0 * w0, axis=1)[None, :]

    x = x_ref[...]
    w = w_ref[...]
    xsq = jnp.sum(x * x, axis=1, keepdims=True)
    wsq = wsq_acc[...]
    s2 = lax.dot_general(x + x, w, (((1,), (1,)), ((), ())),
                         preferred_element_type=jnp.float32)
    d = (xsq - s2) + wsq
    dmin = jnp.min(d, axis=1, keepdims=True)
    iota_f = lax.broadcasted_iota(jnp.int32, (TILE, K), 1).astype(jnp.float32)
    idxf = jnp.min(jnp.where(d == dmin, iota_f, jnp.float32(K)), axis=1,
                   keepdims=True)
    idx_ref[...] = idxf[:, 0].astype(jnp.int32)
    dsum_acc[0] += jnp.sum(dmin)

    @pl.when(step == GRID - 1)
    def _fin():
        dsum_ref[0] = dsum_acc[0]


_argmin_tc = pl.pallas_call(
    _argmin_tc_body,
    grid=(GRID,),
    in_specs=[
        pl.BlockSpec((TILE, D), lambda i: (i, 0)),
        pl.BlockSpec((K, D), lambda i: (0, 0)),
    ],
    out_specs=[
        pl.BlockSpec((TILE,), lambda i: (i,)),
        pl.BlockSpec(memory_space=pltpu.SMEM),
    ],
    out_shape=[
        jax.ShapeDtypeStruct((N,), jnp.int32),
        jax.ShapeDtypeStruct((1,), jnp.float32),
    ],
    scratch_shapes=[
        pltpu.SMEM((1,), jnp.float32),
        pltpu.VMEM((1, K), jnp.float32),
    ],
)


def _scalars_tc_body(idx_ref, dsum_ref, scal_ref, counts_acc):
    step = pl.program_id(0)

    @pl.when(step == 0)
    def _init():
        counts_acc[...] = jnp.zeros_like(counts_acc)

    idx = idx_ref[...]
    iota = lax.broadcasted_iota(jnp.int32, (TILE, K), 1)
    onehot = jnp.where(iota == idx[:, None], 1.0, 0.0)
    counts_acc[...] += jnp.sum(onehot, axis=0, keepdims=True)

    @pl.when(step == GRID - 1)
    def _epilogue():
        counts = counts_acc[...]
        p = counts / jnp.float32(N)
        ent = jnp.sum(p * jnp.log(p + 1e-10))
        perplexity = jnp.exp(-ent)
        usage = jnp.sum((counts > 0.0).astype(jnp.float32)) / jnp.float32(K)
        mse = dsum_ref[0] / jnp.float32(N * D)
        scal_ref[0] = mse + COMMITMENT_COST * mse
        scal_ref[1] = mse
        scal_ref[2] = mse
        scal_ref[3] = perplexity
        scal_ref[4] = usage


_scalars_tc = pl.pallas_call(
    _scalars_tc_body,
    grid=(GRID,),
    in_specs=[
        pl.BlockSpec((TILE,), lambda i: (i,)),
        pl.BlockSpec(memory_space=pltpu.SMEM),
    ],
    out_specs=pl.BlockSpec(memory_space=pltpu.SMEM),
    out_shape=jax.ShapeDtypeStruct((8,), jnp.float32),
    scratch_shapes=[
        pltpu.VMEM((1, K), jnp.float32),
    ],
)


_NC, _NS = 2, 16
_NW = _NC * _NS
_BPW = N // _NW
_CHUNK = 96
_NCHUNK = _BPW // _CHUNK


@functools.cache
def _make_gather():
    @functools.partial(
        pl.kernel,
        mesh=plsc.VectorSubcoreMesh(core_axis_name="c", subcore_axis_name="s",
                                    num_cores=_NC, num_subcores=_NS),
        out_type=jax.ShapeDtypeStruct((N, D), jnp.float32),
        scratch_types=[
            pltpu.VMEM((_BPW,), jnp.int32),
            pltpu.VMEM((_CHUNK, D), jnp.float32),
            pltpu.VMEM((_CHUNK, D), jnp.float32),
            pltpu.SemaphoreType.DMA,
            pltpu.SemaphoreType.DMA,
            pltpu.SemaphoreType.DMA,
            pltpu.SemaphoreType.DMA,
        ],
    )
    def _gather_rows(table_hbm, idx_hbm, out_hbm, idx_v, rows_a, rows_b,
                     gs_a, gs_b, os_a, os_b):
        wid = lax.axis_index("s") * _NC + lax.axis_index("c")
        base = wid * _BPW
        pltpu.sync_copy(idx_hbm.at[pl.ds(base, _BPW)], idx_v)
        rows = (rows_a, rows_b)
        gs = (gs_a, gs_b)
        os = (os_a, os_b)
        g = [None] * _NCHUNK
        o = [None] * _NCHUNK
        for c in range(_NCHUNK):
            b = c & 1
            if c >= 2:
                o[c - 2].wait()
            g[c] = pltpu.async_copy(
                table_hbm.at[idx_v.at[pl.ds(c * _CHUNK, _CHUNK)]],
                rows[b], gs[b])
            if c >= 1:
                g[c - 1].wait()
                o[c - 1] = pltpu.async_copy(
                    rows[(c - 1) & 1],
                    out_hbm.at[pl.ds(base + (c - 1) * _CHUNK, _CHUNK)],
                    os[(c - 1) & 1])
        last = _NCHUNK - 1
        g[last].wait()
        o[last] = pltpu.async_copy(
            rows[last & 1], out_hbm.at[pl.ds(base + last * _CHUNK, _CHUNK)],
            os[last & 1])
        o[last - 1].wait()
        o[last].wait()

    return _gather_rows


def kernel(inputs, weight):
    idx, dsum = _argmin_tc(inputs, weight)
    quantized = _make_gather()(weight, idx)
    scal = _scalars_tc(idx, dsum)
    return (quantized, idx, scal[0], scal[1], scal[2], scal[3], scal[4])


# speedup vs baseline: 1.4333x; 1.1800x over previous
import functools

import jax
import jax.numpy as jnp
from jax import lax
from jax.experimental import pallas as pl
from jax.experimental.pallas import tpu as pltpu
from jax.experimental.pallas import tpu_sc as plsc

K = 1024
D = 256
N = 18432
TILE = 6144
GRID = N // TILE
KC = 512
COMMITMENT_COST = 0.25


def _argmin_tc_body(x_ref, w_ref, idx_ref, dsum_ref, dsum_acc, wsq_acc):
    step = pl.program_id(0)

    @pl.when(step == 0)
    def _init():
        dsum_acc[0] = 0.0
        w0 = w_ref[...]
        wsq_acc[...] = jnp.sum(w0 * w0, axis=1)[None, :]

    x = x_ref[...]
    xsq = jnp.sum(x * x, axis=1, keepdims=True)
    x2 = x + x
    m_run = None
    i_run = None
    for kc in range(K // KC):
        wc = w_ref[pl.ds(kc * KC, KC), :]
        wsqc = wsq_acc[:, pl.ds(kc * KC, KC)]
        s2c = lax.dot_general(x2, wc, (((1,), (1,)), ((), ())),
                              preferred_element_type=jnp.float32)
        dc = (xsq - s2c) + wsqc
        mc = jnp.min(dc, axis=1, keepdims=True)
        iota_f = lax.broadcasted_iota(jnp.int32, (TILE, KC), 1).astype(jnp.float32)
        ic = jnp.min(jnp.where(dc == mc, iota_f, jnp.float32(KC)), axis=1,
                     keepdims=True) + jnp.float32(kc * KC)
        if kc == 0:
            m_run, i_run = mc, ic
        else:
            better = mc < m_run
            i_run = jnp.where(better, ic, i_run)
            m_run = jnp.minimum(m_run, mc)
    idx_ref[...] = i_run[:, 0].astype(jnp.int32)
    dsum_acc[0] += jnp.sum(m_run)

    @pl.when(step == GRID - 1)
    def _fin():
        dsum_ref[0] = dsum_acc[0]


_argmin_tc = pl.pallas_call(
    _argmin_tc_body,
    grid=(GRID,),
    in_specs=[
        pl.BlockSpec((TILE, D), lambda i: (i, 0)),
        pl.BlockSpec((K, D), lambda i: (0, 0)),
    ],
    out_specs=[
        pl.BlockSpec((TILE,), lambda i: (i,)),
        pl.BlockSpec(memory_space=pltpu.SMEM),
    ],
    out_shape=[
        jax.ShapeDtypeStruct((N,), jnp.int32),
        jax.ShapeDtypeStruct((1,), jnp.float32),
    ],
    scratch_shapes=[
        pltpu.SMEM((1,), jnp.float32),
        pltpu.VMEM((1, K), jnp.float32),
    ],
)


def _scalars_tc_body(idx_ref, dsum_ref, scal_ref, counts_acc):
    step = pl.program_id(0)

    @pl.when(step == 0)
    def _init():
        counts_acc[...] = jnp.zeros_like(counts_acc)

    idx = idx_ref[...]
    iota = lax.broadcasted_iota(jnp.int32, (TILE, K), 1)
    onehot = jnp.where(iota == idx[:, None], 1.0, 0.0)
    counts_acc[...] += jnp.sum(onehot, axis=0, keepdims=True)

    @pl.when(step == GRID - 1)
    def _epilogue():
        counts = counts_acc[...]
        p = counts / jnp.float32(N)
        ent = jnp.sum(p * jnp.log(p + 1e-10))
        perplexity = jnp.exp(-ent)
        usage = jnp.sum((counts > 0.0).astype(jnp.float32)) / jnp.float32(K)
        mse = dsum_ref[0] / jnp.float32(N * D)
        scal_ref[0] = mse + COMMITMENT_COST * mse
        scal_ref[1] = mse
        scal_ref[2] = mse
        scal_ref[3] = perplexity
        scal_ref[4] = usage


_scalars_tc = pl.pallas_call(
    _scalars_tc_body,
    grid=(GRID,),
    in_specs=[
        pl.BlockSpec((TILE,), lambda i: (i,)),
        pl.BlockSpec(memory_space=pltpu.SMEM),
    ],
    out_specs=pl.BlockSpec(memory_space=pltpu.SMEM),
    out_shape=jax.ShapeDtypeStruct((8,), jnp.float32),
    scratch_shapes=[
        pltpu.VMEM((1, K), jnp.float32),
    ],
)


_NC, _NS = 2, 16
_NW = _NC * _NS
_BPW = N // _NW
_CHUNK = 96
_NCHUNK = _BPW // _CHUNK


@functools.cache
def _make_gather():
    @functools.partial(
        pl.kernel,
        mesh=plsc.VectorSubcoreMesh(core_axis_name="c", subcore_axis_name="s",
                                    num_cores=_NC, num_subcores=_NS),
        out_type=jax.ShapeDtypeStruct((N, D), jnp.float32),
        scratch_types=[
            pltpu.VMEM((_BPW,), jnp.int32),
            pltpu.VMEM((_CHUNK, D), jnp.float32),
            pltpu.VMEM((_CHUNK, D), jnp.float32),
            pltpu.SemaphoreType.DMA,
            pltpu.SemaphoreType.DMA,
            pltpu.SemaphoreType.DMA,
            pltpu.SemaphoreType.DMA,
        ],
    )
    def _gather_rows(table_hbm, idx_hbm, out_hbm, idx_v, rows_a, rows_b,
                     gs_a, gs_b, os_a, os_b):
        wid = lax.axis_index("s") * _NC + lax.axis_index("c")
        base = wid * _BPW
        pltpu.sync_copy(idx_hbm.at[pl.ds(base, _BPW)], idx_v)
        rows = (rows_a, rows_b)
        gs = (gs_a, gs_b)
        os = (os_a, os_b)
        g = [None] * _NCHUNK
        o = [None] * _NCHUNK
        for c in range(_NCHUNK):
            b = c & 1
            if c >= 2:
                o[c - 2].wait()
            g[c] = pltpu.async_copy(
                table_hbm.at[idx_v.at[pl.ds(c * _CHUNK, _CHUNK)]],
                rows[b], gs[b])
            if c >= 1:
                g[c - 1].wait()
                o[c - 1] = pltpu.async_copy(
                    rows[(c - 1) & 1],
                    out_hbm.at[pl.ds(base + (c - 1) * _CHUNK, _CHUNK)],
                    os[(c - 1) & 1])
        last = _NCHUNK - 1
        g[last].wait()
        o[last] = pltpu.async_copy(
            rows[last & 1], out_hbm.at[pl.ds(base + last * _CHUNK, _CHUNK)],
            os[last & 1])
        o[last - 1].wait()
        o[last].wait()

    return _gather_rows


def kernel(inputs, weight):
    idx, dsum = _argmin_tc(inputs, weight)
    quantized = _make_gather()(weight, idx)
    scal = _scalars_tc(idx, dsum)
    return (quantized, idx, scal[0], scal[1], scal[2], scal[3], scal[4])


# speedup vs baseline: 1.4369x; 1.0025x over previous
import functools

import jax
import jax.numpy as jnp
from jax import lax
from jax.experimental import pallas as pl
from jax.experimental.pallas import tpu as pltpu
from jax.experimental.pallas import tpu_sc as plsc

K = 1024
D = 256
N = 18432
TILE = 6144
GRID = N // TILE
KC = 512
COMMITMENT_COST = 0.25


def _argmin_tc_body(x_ref, w_ref, idx_ref, dsum_ref, dsum_acc, wsq_acc):
    step = pl.program_id(0)

    @pl.when(step == 0)
    def _init():
        dsum_acc[0] = 0.0
        w0 = w_ref[...]
        wsq_acc[...] = jnp.sum(w0 * w0, axis=1)[None, :]

    x = x_ref[...]
    xsq = jnp.sum(x * x, axis=1, keepdims=True)
    x2 = x + x
    m_run = None
    i_run = None
    for kc in range(K // KC):
        wc = w_ref[pl.ds(kc * KC, KC), :]
        wsqc = wsq_acc[:, pl.ds(kc * KC, KC)]
        s2c = lax.dot_general(x2, wc, (((1,), (1,)), ((), ())),
                              preferred_element_type=jnp.float32)
        dc = (xsq - s2c) + wsqc
        mc = jnp.min(dc, axis=1, keepdims=True)
        iota_f = lax.broadcasted_iota(jnp.int32, (TILE, KC), 1).astype(jnp.float32)
        ic = jnp.min(jnp.where(dc == mc, iota_f, jnp.float32(KC)), axis=1,
                     keepdims=True) + jnp.float32(kc * KC)
        if kc == 0:
            m_run, i_run = mc, ic
        else:
            better = mc < m_run
            i_run = jnp.where(better, ic, i_run)
            m_run = jnp.minimum(m_run, mc)
    idx_ref[...] = i_run[:, 0].astype(jnp.int32)
    dsum_acc[0] += jnp.sum(m_run)

    @pl.when(step == GRID - 1)
    def _fin():
        dsum_ref[0] = dsum_acc[0]


_argmin_tc = pl.pallas_call(
    _argmin_tc_body,
    grid=(GRID,),
    in_specs=[
        pl.BlockSpec((TILE, D), lambda i: (i, 0)),
        pl.BlockSpec((K, D), lambda i: (0, 0)),
    ],
    out_specs=[
        pl.BlockSpec((TILE,), lambda i: (i,)),
        pl.BlockSpec(memory_space=pltpu.SMEM),
    ],
    out_shape=[
        jax.ShapeDtypeStruct((N,), jnp.int32),
        jax.ShapeDtypeStruct((1,), jnp.float32),
    ],
    scratch_shapes=[
        pltpu.SMEM((1,), jnp.float32),
        pltpu.VMEM((1, K), jnp.float32),
    ],
)


def _scalars_tc_body(idx_ref, dsum_ref, scal_ref, counts_acc):
    step = pl.program_id(0)

    @pl.when(step == 0)
    def _init():
        counts_acc[...] = jnp.zeros_like(counts_acc)

    idx = idx_ref[...]
    iota = lax.broadcasted_iota(jnp.int32, (TILE, K), 1)
    onehot = jnp.where(iota == idx[:, None], 1.0, 0.0)
    counts_acc[...] += jnp.sum(onehot, axis=0, keepdims=True)

    @pl.when(step == GRID - 1)
    def _epilogue():
        counts = counts_acc[...]
        p = counts / jnp.float32(N)
        ent = jnp.sum(p * jnp.log(p + 1e-10))
        perplexity = jnp.exp(-ent)
        usage = jnp.sum((counts > 0.0).astype(jnp.float32)) / jnp.float32(K)
        mse = dsum_ref[0] / jnp.float32(N * D)
        scal_ref[0] = mse + COMMITMENT_COST * mse
        scal_ref[1] = mse
        scal_ref[2] = mse
        scal_ref[3] = perplexity
        scal_ref[4] = usage


_scalars_tc = pl.pallas_call(
    _scalars_tc_body,
    grid=(GRID,),
    in_specs=[
        pl.BlockSpec((TILE,), lambda i: (i,)),
        pl.BlockSpec(memory_space=pltpu.SMEM),
    ],
    out_specs=pl.BlockSpec(memory_space=pltpu.SMEM),
    out_shape=jax.ShapeDtypeStruct((8,), jnp.float32),
    scratch_shapes=[
        pltpu.VMEM((1, K), jnp.float32),
    ],
)


_NC, _NS = 2, 16
_NW = _NC * _NS
_BPW = N // _NW
_CHUNK = 96
_NCHUNK = _BPW // _CHUNK
_NBUF = 3


@functools.cache
def _make_gather():
    @functools.partial(
        pl.kernel,
        mesh=plsc.VectorSubcoreMesh(core_axis_name="c", subcore_axis_name="s",
                                    num_cores=_NC, num_subcores=_NS),
        out_type=jax.ShapeDtypeStruct((N, D), jnp.float32),
        scratch_types=[
            pltpu.VMEM((_BPW,), jnp.int32),
        ] + [pltpu.VMEM((_CHUNK, D), jnp.float32)] * _NBUF
          + [pltpu.SemaphoreType.DMA] * (2 * _NBUF),
    )
    def _gather_rows(table_hbm, idx_hbm, out_hbm, idx_v, *bufs):
        rows = bufs[:_NBUF]
        gs = bufs[_NBUF:2 * _NBUF]
        os = bufs[2 * _NBUF:]
        wid = lax.axis_index("s") * _NC + lax.axis_index("c")
        base = wid * _BPW
        pltpu.sync_copy(idx_hbm.at[pl.ds(base, _BPW)], idx_v)
        g = [None] * _NCHUNK
        o = [None] * _NCHUNK
        for c in range(_NCHUNK):
            b = c % _NBUF
            if c >= _NBUF:
                o[c - _NBUF].wait()
            g[c] = pltpu.async_copy(
                table_hbm.at[idx_v.at[pl.ds(c * _CHUNK, _CHUNK)]],
                rows[b], gs[b])
            if c >= 1:
                g[c - 1].wait()
                o[c - 1] = pltpu.async_copy(
                    rows[(c - 1) % _NBUF],
                    out_hbm.at[pl.ds(base + (c - 1) * _CHUNK, _CHUNK)],
                    os[(c - 1) % _NBUF])
        last = _NCHUNK - 1
        g[last].wait()
        o[last] = pltpu.async_copy(
            rows[last % _NBUF], out_hbm.at[pl.ds(base + last * _CHUNK, _CHUNK)],
            os[last % _NBUF])
        for c in range(max(0, _NCHUNK - _NBUF), _NCHUNK):
            o[c].wait()

    return _gather_rows


def kernel(inputs, weight):
    idx, dsum = _argmin_tc(inputs, weight)
    quantized = _make_gather()(weight, idx)
    scal = _scalars_tc(idx, dsum)
    return (quantized, idx, scal[0], scal[1], scal[2], scal[3], scal[4])
